# baseline (speedup 1.0000x reference)
import functools
import math

import jax
import jax.numpy as jnp
from jax import lax
from jax.experimental import pallas as pl
from jax.experimental.pallas import tpu as pltpu
from jax.experimental.pallas import tpu_sc as plsc

D = 64
SCALE = math.sqrt(D)
NC, NS, L = 2, 16, 16
NW = NC * NS
U = 128
B_ROWS = 4096
SEQ = 200
TOTAL = B_ROWS * SEQ
UNITS = TOTAL // U
UPW = UNITS // NW
VT = 1000000 // 2


@functools.partial(
    pl.kernel,
    mesh=plsc.VectorSubcoreMesh(core_axis_name="c", subcore_axis_name="s"),
    compiler_params=pltpu.CompilerParams(use_tc_tiling_on_sc=False),
    out_type=jax.ShapeDtypeStruct((TOTAL * D // 128, 128), jnp.float32),
    scratch_types=[
        pltpu.VMEM((UPW, U), jnp.int32),
        pltpu.VMEM((U,), jnp.int32),
        pltpu.VMEM((U, 128), jnp.float32),
        pltpu.VMEM((U // 2, 128), jnp.float32),
        pltpu.SemaphoreType.DMA,
    ],
)
def _emb_lookup(x_hbm, table_hbm, out_hbm, idx_v, idx2_v, buf_v, cbuf_v, sem):
    c = lax.axis_index("c")
    s = lax.axis_index("s")
    wid = s * NC + c
    pltpu.sync_copy(x_hbm.at[pl.ds(wid * UPW, UPW)], idx_v)

    def unit(u, carry):
        def halve(g, carry2):
            iv = idx_v[u, pl.ds(g * L, L)]
            idx2_v[pl.ds(g * L, L)] = lax.shift_right_logical(iv, 1)
            return carry2

        lax.fori_loop(0, U // L, halve, 0)
        pltpu.async_copy(table_hbm.at[idx2_v], buf_v, sem).wait()

        def group(g, carry2):
            iv = idx_v[u, pl.ds(g * L, L)]
            pv = iv & 1

            def row(r, carry3):
                rr = g * L + r
                pb = pv.at[jnp.full((L,), r, jnp.int32)].get(
                    mode="promise_in_bounds")
                pf = pb.astype(jnp.float32)
                for j in range(D // L):
                    left = buf_v[rr, pl.ds(j * L, L)]
                    right = buf_v[rr, pl.ds(D + j * L, L)]
                    v = (left + (right - left) * pf) * SCALE
                    half = rr & 1
                    cbuf_v[lax.shift_right_logical(rr, 1),
                           pl.ds(half * D + j * L, L)] = v
                return carry3

            lax.fori_loop(0, L, row, 0)
            return carry2

        lax.fori_loop(0, U // L, group, 0)

        pltpu.sync_copy(cbuf_v, out_hbm.at[pl.ds((wid * UPW + u) * (U // 2),
                                                 U // 2)])
        return carry

    lax.fori_loop(0, UPW, unit, 0)


def kernel(x, table):
    x2 = x.reshape(UNITS, U).astype(jnp.int32)
    t2 = table.reshape(VT, 128)
    out = _emb_lookup(x2, t2)
    return out.reshape(B_ROWS, SEQ, D)


# speedup vs baseline: 1.0029x; 1.0029x over previous
import functools
import math

import jax
import jax.numpy as jnp
from jax import lax
from jax.experimental import pallas as pl
from jax.experimental.pallas import tpu as pltpu
from jax.experimental.pallas import tpu_sc as plsc

D = 64
SCALE = math.sqrt(D)
NC, NS, L = 2, 16, 16
NW = NC * NS
U = 128
B_ROWS = 4096
SEQ = 200
TOTAL = B_ROWS * SEQ
UNITS = TOTAL // U
UPW = UNITS // NW
VT = 1000000 // 2


@functools.partial(
    pl.kernel,
    mesh=plsc.VectorSubcoreMesh(core_axis_name="c", subcore_axis_name="s"),
    out_type=jax.ShapeDtypeStruct((TOTAL * D // 128, 128), jnp.float32),
    scratch_types=[
        pltpu.VMEM((UPW, U), jnp.int32),
        pltpu.VMEM((U,), jnp.int32),
        pltpu.VMEM((U, 128), jnp.float32),
        pltpu.VMEM((U // 2, 128), jnp.float32),
        pltpu.SemaphoreType.DMA,
    ],
)
def _emb_lookup(x_hbm, table_hbm, out_hbm, idx_v, idx2_v, buf_v, cbuf_v, sem):
    c = lax.axis_index("c")
    s = lax.axis_index("s")
    wid = s * NC + c
    pltpu.sync_copy(x_hbm.at[pl.ds(wid * UPW, UPW)], idx_v)

    def unit(u, carry):
        def halve(g, carry2):
            iv = idx_v[u, pl.ds(g * L, L)]
            idx2_v[pl.ds(g * L, L)] = lax.shift_right_logical(iv, 1)
            return carry2

        lax.fori_loop(0, U // L, halve, 0)
        pltpu.async_copy(table_hbm.at[idx2_v], buf_v, sem).wait()

        def group(g, carry2):
            iv = idx_v[u, pl.ds(g * L, L)]
            pv = iv & 1

            def row(r, carry3):
                rr = g * L + r
                pb = pv.at[jnp.full((L,), r, jnp.int32)].get(
                    mode="promise_in_bounds")
                pf = pb.astype(jnp.float32)
                for j in range(D // L):
                    left = buf_v[rr, pl.ds(j * L, L)]
                    right = buf_v[rr, pl.ds(D + j * L, L)]
                    v = (left + (right - left) * pf) * SCALE
                    half = rr & 1
                    cbuf_v[lax.shift_right_logical(rr, 1),
                           pl.ds(half * D + j * L, L)] = v
                return carry3

            lax.fori_loop(0, L, row, 0)
            return carry2

        lax.fori_loop(0, U // L, group, 0)

        pltpu.sync_copy(cbuf_v, out_hbm.at[pl.ds((wid * UPW + u) * (U // 2),
                                                 U // 2)])
        return carry

    lax.fori_loop(0, UPW, unit, 0)


def kernel(x, table):
    x2 = x.reshape(UNITS, U).astype(jnp.int32)
    t2 = table.reshape(VT, 128)
    out = _emb_lookup(x2, t2)
    return out.reshape(B_ROWS, SEQ, D)


# speedup vs baseline: 1.6343x; 1.6295x over previous
import functools
import math

import jax
import jax.numpy as jnp
from jax import lax
from jax.experimental import pallas as pl
from jax.experimental.pallas import tpu as pltpu
from jax.experimental.pallas import tpu_sc as plsc

D = 64
SCALE = math.sqrt(D)
NC, NS, L = 2, 16, 16
NW = NC * NS
B_ROWS = 4096
SEQ = 200
BPW = B_ROWS // NW


@functools.partial(
    pl.kernel,
    mesh=plsc.VectorSubcoreMesh(core_axis_name="c", subcore_axis_name="s"),
    compiler_params=pltpu.CompilerParams(use_tc_tiling_on_sc=False),
    out_type=jax.ShapeDtypeStruct((B_ROWS, SEQ, D), jnp.float32),
    scratch_types=[
        pltpu.VMEM((BPW, SEQ), jnp.int32),
        pltpu.VMEM((SEQ, D), jnp.float32),
        pltpu.SemaphoreType.DMA,
    ],
)
def _emb_lookup(x_hbm, table_hbm, out_hbm, idx_v, rows_v, sem):
    c = lax.axis_index("c")
    s = lax.axis_index("s")
    wid = s * NC + c
    b0 = wid * BPW
    pltpu.sync_copy(x_hbm.at[pl.ds(b0, BPW)], idx_v)

    def brow(i, carry):
        cp1 = pltpu.async_copy(table_hbm.at[idx_v.at[i, pl.ds(0, 128)]],
                               rows_v.at[pl.ds(0, 128)], sem)
        cp2 = pltpu.async_copy(table_hbm.at[idx_v.at[i, pl.ds(128, SEQ - 128)]],
                               rows_v.at[pl.ds(128, SEQ - 128)], sem)
        cp1.wait()
        cp2.wait()

        def mul_row(r, carry2):
            for j in range(D // L):
                sl = rows_v[r, pl.ds(j * L, L)]
                rows_v[r, pl.ds(j * L, L)] = sl * SCALE
            return carry2

        lax.fori_loop(0, SEQ, mul_row, 0)
        pltpu.sync_copy(rows_v, out_hbm.at[b0 + i])
        return carry

    lax.fori_loop(0, BPW, brow, 0)


def kernel(x, table):
    return _emb_lookup(x.astype(jnp.int32), table)
